# baseline (speedup 1.0000x reference)
import functools

import jax
import jax.numpy as jnp
import numpy as np
from jax import lax
from jax.experimental import pallas as pl
from jax.experimental.pallas import tpu as pltpu
from jax.experimental.pallas import tpu_sc as plsc

N = 10000
E = 160000
H = 128
HALF = 64
Z = 32

_BM_N = 2000
_BM_E = 2000

_NSC = 2
_NTEC = 16
_EPT = E // _NTEC
_C = 80
_NCH = _EPT // _C
_NCOPY = 10
_NSTRIPE = N // _NCOPY

_F32 = jnp.float32


def _linear_dense(x, w, b, *, relu=False, block_m):
    M, K = x.shape
    F = w.shape[1]

    def body(x_ref, w_ref, b_ref, o_ref):
        acc = jnp.dot(x_ref[...], w_ref[...], preferred_element_type=_F32)
        acc = acc + b_ref[...]
        if relu:
            acc = jnp.maximum(acc, 0.0)
        o_ref[...] = acc

    return pl.pallas_call(
        body,
        grid=(M // block_m,),
        in_specs=[
            pl.BlockSpec((block_m, K), lambda i: (i, 0)),
            pl.BlockSpec((K, F), lambda i: (0, 0)),
            pl.BlockSpec((1, F), lambda i: (0, 0)),
        ],
        out_specs=pl.BlockSpec((block_m, F), lambda i: (i, 0)),
        out_shape=jax.ShapeDtypeStruct((M, F), _F32),
    )(x, w, b.reshape(1, F))


def _abde(h, w_all, b_all):
    def body(x_ref, w_ref, b_ref, a_ref, db_ref, e2_ref):
        acc = jnp.dot(x_ref[...], w_ref[...], preferred_element_type=_F32)
        acc = acc + b_ref[...]
        a_ref[...] = acc[:, 0:128]
        db_ref[0] = jnp.concatenate(
            [acc[:, 256:320], acc[:, 128:192]], axis=1)
        db_ref[1] = jnp.concatenate(
            [acc[:, 320:384], acc[:, 192:256]], axis=1)
        e2_ref[0] = acc[:, 384:448]
        e2_ref[1] = acc[:, 448:512]

    return pl.pallas_call(
        body,
        grid=(N // _BM_N,),
        in_specs=[
            pl.BlockSpec((_BM_N, H), lambda i: (i, 0)),
            pl.BlockSpec((H, 512), lambda i: (0, 0)),
            pl.BlockSpec((1, 512), lambda i: (0, 0)),
        ],
        out_specs=[
            pl.BlockSpec((_BM_N, H), lambda i: (i, 0)),
            pl.BlockSpec((2, _BM_N, H), lambda i: (0, i, 0)),
            pl.BlockSpec((2, _BM_N, HALF), lambda i: (0, i, 0)),
        ],
        out_shape=[
            jax.ShapeDtypeStruct((N, H), _F32),
            jax.ShapeDtypeStruct((2, N, H), _F32),
            jax.ShapeDtypeStruct((2, N, HALF), _F32),
        ],
    )(h, w_all, b_all)


def _ce_pack(e2, w, b):
    def body(x_ref, w_ref, b_ref, o_ref):
        x = jnp.concatenate([x_ref[0], x_ref[1]], axis=1)
        acc = jnp.dot(x, w_ref[...], preferred_element_type=_F32)
        acc = acc + b_ref[...]
        o_ref[0] = jnp.concatenate([acc[:, 0:64], x_ref[0]], axis=1)
        o_ref[1] = jnp.concatenate([acc[:, 64:128], x_ref[1]], axis=1)

    return pl.pallas_call(
        body,
        grid=(E // _BM_E,),
        in_specs=[
            pl.BlockSpec((2, _BM_E, HALF), lambda i: (0, i, 0)),
            pl.BlockSpec((H, H), lambda i: (0, 0)),
            pl.BlockSpec((1, H), lambda i: (0, 0)),
        ],
        out_specs=pl.BlockSpec((2, _BM_E, H), lambda i: (0, i, 0)),
        out_shape=jax.ShapeDtypeStruct((2, E, H), _F32),
    )(e2, w, b.reshape(1, H))


def _ce_plain(e2, w, b):
    def body(x_ref, w_ref, b_ref, o_ref):
        x = jnp.concatenate([x_ref[0], x_ref[1]], axis=1)
        acc = jnp.dot(x, w_ref[...], preferred_element_type=_F32)
        acc = acc + b_ref[...]
        o_ref[0] = acc[:, 0:64]
        o_ref[1] = acc[:, 64:128]

    return pl.pallas_call(
        body,
        grid=(E // _BM_E,),
        in_specs=[
            pl.BlockSpec((2, _BM_E, HALF), lambda i: (0, i, 0)),
            pl.BlockSpec((H, H), lambda i: (0, 0)),
            pl.BlockSpec((1, H), lambda i: (0, 0)),
        ],
        out_specs=pl.BlockSpec((2, _BM_E, HALF), lambda i: (0, i, 0)),
        out_shape=jax.ShapeDtypeStruct((2, E, HALF), _F32),
    )(e2, w, b.reshape(1, H))


def _emb_e(ex, w, b):
    K = ex.shape[1]

    def body(x_ref, w_ref, b_ref, o_ref):
        acc = jnp.dot(x_ref[...], w_ref[...], preferred_element_type=_F32)
        acc = acc + b_ref[...]
        o_ref[0] = acc[:, 0:64]
        o_ref[1] = acc[:, 64:128]

    return pl.pallas_call(
        body,
        grid=(E // _BM_E,),
        in_specs=[
            pl.BlockSpec((_BM_E, K), lambda i: (i, 0)),
            pl.BlockSpec((K, H), lambda i: (0, 0)),
            pl.BlockSpec((1, H), lambda i: (0, 0)),
        ],
        out_specs=pl.BlockSpec((2, _BM_E, HALF), lambda i: (0, i, 0)),
        out_shape=jax.ShapeDtypeStruct((2, E, HALF), _F32),
    )(ex, w, b.reshape(1, H))


def _node_update(h_in, Ah, nd2, g, b):
    def body(h_ref, a_ref, nd_ref, g_ref, b_ref, o_ref):
        r0 = nd_ref[0][:, 0:HALF] / (nd_ref[0][:, HALF:H] + 1e-6)
        r1 = nd_ref[1][:, 0:HALF] / (nd_ref[1][:, HALF:H] + 1e-6)
        hn = a_ref[...] + jnp.concatenate([r0, r1], axis=1)
        hn = jnp.maximum(hn * g_ref[...] + b_ref[...], 0.0)
        o_ref[...] = h_ref[...] + hn

    return pl.pallas_call(
        body,
        grid=(N // _BM_N,),
        in_specs=[
            pl.BlockSpec((_BM_N, H), lambda i: (i, 0)),
            pl.BlockSpec((_BM_N, H), lambda i: (i, 0)),
            pl.BlockSpec((2, _BM_N, H), lambda i: (0, i, 0)),
            pl.BlockSpec((1, H), lambda i: (0, 0)),
            pl.BlockSpec((1, H), lambda i: (0, 0)),
        ],
        out_specs=pl.BlockSpec((_BM_N, H), lambda i: (i, 0)),
        out_shape=jax.ShapeDtypeStruct((N, H), _F32),
    )(h_in, Ah, nd2, g, b)


def _dec_emb(h1, lat, eps, wh, wz, b):
    def body(h_ref, l_ref, e_ref, wh_ref, wz_ref, b_ref, o_ref):
        mean = l_ref[:, 0:Z]
        logvar = l_ref[:, Z:2 * Z]
        z = e_ref[...] * jnp.exp(0.5 * logvar) + mean
        acc = jnp.dot(h_ref[...], wh_ref[...], preferred_element_type=_F32)
        acc = acc + jnp.dot(z, wz_ref[...], preferred_element_type=_F32)
        o_ref[...] = acc + b_ref[...]

    return pl.pallas_call(
        body,
        grid=(N // _BM_N,),
        in_specs=[
            pl.BlockSpec((_BM_N, H), lambda i: (i, 0)),
            pl.BlockSpec((_BM_N, 2 * Z), lambda i: (i, 0)),
            pl.BlockSpec((_BM_N, Z), lambda i: (i, 0)),
            pl.BlockSpec((H, H), lambda i: (0, 0)),
            pl.BlockSpec((Z, H), lambda i: (0, 0)),
            pl.BlockSpec((1, H), lambda i: (0, 0)),
        ],
        out_specs=pl.BlockSpec((_BM_N, H), lambda i: (i, 0)),
        out_shape=jax.ShapeDtypeStruct((N, H), _F32),
    )(h1, lat, eps, wh, wz, b.reshape(1, H))


def _readout(h, ro):
    w1, b1 = ro[0]["w"], ro[0]["b"]
    w2, b2 = ro[1]["w"], ro[1]["b"]
    w3, b3 = ro[2]["w"], ro[2]["b"]

    def body(x_ref, w1r, b1r, w2r, b2r, w3r, b3r, o_ref):
        y = jnp.dot(x_ref[...], w1r[...], preferred_element_type=_F32) + b1r[...]
        y = jnp.maximum(y, 0.0)
        y = jnp.dot(y, w2r[...], preferred_element_type=_F32) + b2r[...]
        y = jnp.maximum(y, 0.0)
        o_ref[...] = jnp.dot(y, w3r[...], preferred_element_type=_F32) + b3r[...]

    return pl.pallas_call(
        body,
        grid=(N // _BM_N,),
        in_specs=[
            pl.BlockSpec((_BM_N, H), lambda i: (i, 0)),
            pl.BlockSpec((H, 64), lambda i: (0, 0)),
            pl.BlockSpec((1, 64), lambda i: (0, 0)),
            pl.BlockSpec((64, 32), lambda i: (0, 0)),
            pl.BlockSpec((1, 32), lambda i: (0, 0)),
            pl.BlockSpec((32, 2), lambda i: (0, 0)),
            pl.BlockSpec((1, 2), lambda i: (0, 0)),
        ],
        out_specs=pl.BlockSpec((_BM_N, 2), lambda i: (i, 0)),
        out_shape=jax.ShapeDtypeStruct((N, 2), _F32),
    )(h, w1, b1.reshape(1, 64), w2, b2.reshape(1, 32), w3, b3.reshape(1, 2))


def _concat_e(e2):
    def body(x_ref, o_ref):
        o_ref[...] = jnp.concatenate([x_ref[0], x_ref[1]], axis=1)

    return pl.pallas_call(
        body,
        grid=(E // _BM_E,),
        in_specs=[pl.BlockSpec((2, _BM_E, HALF), lambda i: (0, i, 0))],
        out_specs=pl.BlockSpec((_BM_E, H), lambda i: (i, 0)),
        out_shape=jax.ShapeDtypeStruct((E, H), _F32),
    )(e2)


def _make_sc_edge():
    mesh = plsc.VectorSubcoreMesh(core_axis_name="c", subcore_axis_name="s",
                                  num_cores=_NSC, num_subcores=_NTEC)
    @functools.partial(
        pl.kernel,
        out_type=[
            jax.ShapeDtypeStruct((2 * E, HALF), _F32),
            jax.ShapeDtypeStruct((2 * N, H), _F32),
        ],
        mesh=mesh,
        compiler_params=pltpu.CompilerParams(use_tc_tiling_on_sc=False),
        scratch_types=[
            pltpu.VMEM((2, _C), jnp.int32),
            pltpu.VMEM((2, _C), jnp.int32),
            pltpu.VMEM((2, _C), jnp.int32),
            pltpu.VMEM((2, _C), jnp.int32),
            pltpu.VMEM((_C, H), _F32),
            pltpu.VMEM((_C, HALF), _F32),
            pltpu.VMEM((_C, H), _F32),
            pltpu.VMEM((_C, HALF), _F32),
            pltpu.VMEM((_C, H), _F32),
            pltpu.VMEM((HALF,), _F32),
            pltpu.VMEM((HALF,), _F32),
            pltpu.VMEM_SHARED((N, H), _F32),
            pltpu.SemaphoreType.DMA,
            pltpu.SemaphoreType.DMA,
            pltpu.SemaphoreType.DMA,
        ],
    )
    def sck(dbtab_h, etab_h, cei_h, src_h, dst_h, g_h, b_h,
            eout_h, nd_h,
            sraw, draw, sadj, dadj, db, eh, cei, eo, msgsg,
            gv, bv, nd_sh,
            isem0, isem1, lsem):
        isem = (isem0, isem1)
        c = lax.axis_index("c")
        s = lax.axis_index("s")
        cn = c * N
        ebase = s * _EPT

        pltpu.sync_copy(g_h.at[pl.ds(c * HALF, HALF)], gv)
        pltpu.sync_copy(b_h.at[pl.ds(c * HALF, HALF)], bv)

        zv = jnp.zeros((16,), _F32)

        def zrow2(r, carry):
            for j in range(H // 16):
                msgsg[r, pl.ds(j * 16, 16)] = zv
            return carry

        lax.fori_loop(0, _C, zrow2, 0)

        @pl.when(s < _NCOPY)
        def _zero_stripes():
            for q in range(_NSTRIPE // _C):
                off = s * _NSTRIPE + q * _C
                pltpu.sync_copy(msgsg, nd_sh.at[pl.ds(off, _C)])
            rem = _NSTRIPE - (_NSTRIPE // _C) * _C
            if rem:
                off = s * _NSTRIPE + (_NSTRIPE // _C) * _C
                pltpu.sync_copy(msgsg.at[pl.ds(0, rem)],
                                nd_sh.at[pl.ds(off, rem)])

        plsc.subcore_barrier()

        def issue_idx(k, r):
            base = ebase + k * _C
            pltpu.async_copy(src_h.at[pl.ds(base, _C)], sraw.at[r], isem[r])
            pltpu.async_copy(dst_h.at[pl.ds(base, _C)], draw.at[r], isem[r])

        def wait_idx_adjust(r):
            pltpu.make_async_copy(
                src_h.at[pl.ds(0, _C)], sraw.at[r], isem[r]).wait()
            pltpu.make_async_copy(
                dst_h.at[pl.ds(0, _C)], draw.at[r], isem[r]).wait()
            for j in range(_C // 16):
                sl = pl.ds(j * 16, 16)
                sadj[r, sl] = sraw[r, sl] + cn
                dadj[r, sl] = draw[r, sl] + cn

        def compute():
            gs = tuple(gv[pl.ds(j * 16, 16)] for j in range(HALF // 16))
            bs = tuple(bv[pl.ds(j * 16, 16)] for j in range(HALF // 16))

            def row(r, carry):
                cgs, cbs = carry
                for j in range(HALF // 16):
                    sl = pl.ds(j * 16, 16)
                    slb = pl.ds(HALF + j * 16, 16)
                    en = db[r, sl] + eh[r, sl] + cei[r, sl]
                    sgv = 1.0 / (1.0 + jnp.exp(-en))
                    msgsg[r, sl] = sgv * db[r, slb]
                    msgsg[r, slb] = sgv
                    eo[r, sl] = (
                        jnp.maximum(en * cgs[j] + cbs[j], 0.0) + cei[r, slb])
                return carry

            lax.fori_loop(0, _C, row, (gs, bs))

        def do_chunk(k, r):
            fb = c * E + ebase + k * _C
            l1 = pltpu.async_copy(dbtab_h.at[sadj.at[r]], db, lsem)
            l2 = pltpu.async_copy(etab_h.at[dadj.at[r]], eh, lsem)
            l3 = pltpu.async_copy(cei_h.at[pl.ds(fb, _C)], cei, lsem)
            l1.wait()
            l2.wait()
            l3.wait()
            compute()
            pltpu.sync_copy(eo, eout_h.at[pl.ds(fb, _C)])
            pltpu.sync_copy(msgsg, nd_sh.at[draw.at[r]], add=True)

        issue_idx(0, 0)

        def dbl(i, carry):
            k0 = 2 * i
            wait_idx_adjust(0)
            issue_idx(k0 + 1, 1)
            do_chunk(k0, 0)
            wait_idx_adjust(1)
            issue_idx(k0 + 2, 0)
            do_chunk(k0 + 1, 1)
            return carry

        lax.fori_loop(0, (_NCH - 1) // 2, dbl, 0)

        wait_idx_adjust(0)
        do_chunk(_NCH - 1, 0)

        plsc.subcore_barrier()

        @pl.when(s < _NCOPY)
        def _copy_out():
            nb = cn + s * _NSTRIPE
            pltpu.sync_copy(nd_sh.at[pl.ds(s * _NSTRIPE, _NSTRIPE)],
                            nd_h.at[pl.ds(nb, _NSTRIPE)])

    return sck


_SC_EDGE = _make_sc_edge()


def _sc_edge(DB2, E2t, CEI2, src, dst, g, b):
    eo, nd = _SC_EDGE(
        DB2.reshape(2 * N, H), E2t.reshape(2 * N, HALF),
        CEI2.reshape(2 * E, H), src, dst, g, b)
    return eo.reshape(2, E, HALF), nd.reshape(2, N, H)


_BN_SCALE = 1.0 / np.sqrt(1.0 + 1e-5)


def _layer(h, e2, src, dst, lp):
    w_all = jnp.concatenate(
        [lp["A"]["w"], lp["B"]["w"], lp["D"]["w"], lp["E"]["w"]], axis=1)
    b_all = jnp.concatenate(
        [lp["A"]["b"], lp["B"]["b"], lp["D"]["b"], lp["E"]["b"]]).reshape(1, 512)
    Ah, DB2, E2t = _abde(h, w_all, b_all)
    CEI2 = _ce_pack(e2, lp["C"]["w"], lp["C"]["b"])
    ge = lp["bn_e_g"] * _BN_SCALE
    eout2, nd2 = _sc_edge(DB2, E2t, CEI2, src, dst, ge, lp["bn_e_b"])
    gh = (lp["bn_h_g"] * _BN_SCALE).reshape(1, H)
    h = _node_update(h, Ah, nd2, gh, lp["bn_h_b"].reshape(1, H))
    return h, eout2


def kernel(xx, ex, edge_index, params):
    src = edge_index[0]
    dst = edge_index[1]
    eps = jax.random.normal(jax.random.key(1), (N, Z), dtype=_F32)

    penc = params["past_enc"]
    pdec = params["past_dec"]

    h = _linear_dense(xx, penc["emb_h"]["w"], penc["emb_h"]["b"],
                      block_m=_BM_N)
    e2 = _emb_e(ex, penc["emb_e"]["w"], penc["emb_e"]["b"])
    for lp in penc["layers"]:
        h, e2 = _layer(h, e2, src, dst, lp)

    lat = _linear_dense(h, params["prior"]["w"], params["prior"]["b"],
                        block_m=_BM_N)
    h = _dec_emb(h, lat, eps, pdec["emb_h"]["w"][:H], pdec["emb_h"]["w"][H:],
                 pdec["emb_h"]["b"])
    e2 = _ce_plain(e2, pdec["emb_e"]["w"], pdec["emb_e"]["b"])
    for lp in pdec["layers"]:
        h, e2 = _layer(h, e2, src, dst, lp)

    h_out = _readout(h, pdec["readout"])
    e_out = _concat_e(e2)
    return h_out, e_out


# speedup vs baseline: 1.3271x; 1.3271x over previous
import functools

import jax
import jax.numpy as jnp
import numpy as np
from jax import lax
from jax.experimental import pallas as pl
from jax.experimental.pallas import tpu as pltpu
from jax.experimental.pallas import tpu_sc as plsc

N = 10000
E = 160000
H = 128
HALF = 64
Z = 32

_BM_N = 2000
_BM_E = 2000

_NSC = 2
_NTEC = 16
_EPT = E // _NTEC
_C = 80
_NCH = _EPT // _C
_NCOPY = 10
_NSTRIPE = N // _NCOPY
_ZROWS = 200

_F32 = jnp.float32


def _linear_dense(x, w, b, *, relu=False, block_m):
    M, K = x.shape
    F = w.shape[1]

    def body(x_ref, w_ref, b_ref, o_ref):
        acc = jnp.dot(x_ref[...], w_ref[...], preferred_element_type=_F32)
        acc = acc + b_ref[...]
        if relu:
            acc = jnp.maximum(acc, 0.0)
        o_ref[...] = acc

    return pl.pallas_call(
        body,
        grid=(M // block_m,),
        in_specs=[
            pl.BlockSpec((block_m, K), lambda i: (i, 0)),
            pl.BlockSpec((K, F), lambda i: (0, 0)),
            pl.BlockSpec((1, F), lambda i: (0, 0)),
        ],
        out_specs=pl.BlockSpec((block_m, F), lambda i: (i, 0)),
        out_shape=jax.ShapeDtypeStruct((M, F), _F32),
    )(x, w, b.reshape(1, F))


def _abde(h, w_all, b_all):
    def body(x_ref, w_ref, b_ref, a_ref, b2_ref, d2_ref, e2_ref):
        acc = jnp.dot(x_ref[...], w_ref[...], preferred_element_type=_F32)
        acc = acc + b_ref[...]
        a_ref[...] = acc[:, 0:128]
        for t, ref in enumerate((b2_ref, d2_ref, e2_ref)):
            off = 128 * (t + 1)
            ref[0] = acc[:, off:off + 64]
            ref[1] = acc[:, off + 64:off + 128]

    return pl.pallas_call(
        body,
        grid=(N // _BM_N,),
        in_specs=[
            pl.BlockSpec((_BM_N, H), lambda i: (i, 0)),
            pl.BlockSpec((H, 512), lambda i: (0, 0)),
            pl.BlockSpec((1, 512), lambda i: (0, 0)),
        ],
        out_specs=[
            pl.BlockSpec((_BM_N, H), lambda i: (i, 0)),
            pl.BlockSpec((2, _BM_N, HALF), lambda i: (0, i, 0)),
            pl.BlockSpec((2, _BM_N, HALF), lambda i: (0, i, 0)),
            pl.BlockSpec((2, _BM_N, HALF), lambda i: (0, i, 0)),
        ],
        out_shape=[
            jax.ShapeDtypeStruct((N, H), _F32),
            jax.ShapeDtypeStruct((2, N, HALF), _F32),
            jax.ShapeDtypeStruct((2, N, HALF), _F32),
            jax.ShapeDtypeStruct((2, N, HALF), _F32),
        ],
    )(h, w_all, b_all)


def _ce(e2, w, b):
    def body(x_ref, w_ref, b_ref, o_ref):
        x = jnp.concatenate([x_ref[0], x_ref[1]], axis=1)
        acc = jnp.dot(x, w_ref[...], preferred_element_type=_F32)
        acc = acc + b_ref[...]
        o_ref[0] = acc[:, 0:64]
        o_ref[1] = acc[:, 64:128]

    return pl.pallas_call(
        body,
        grid=(E // _BM_E,),
        in_specs=[
            pl.BlockSpec((2, _BM_E, HALF), lambda i: (0, i, 0)),
            pl.BlockSpec((H, H), lambda i: (0, 0)),
            pl.BlockSpec((1, H), lambda i: (0, 0)),
        ],
        out_specs=pl.BlockSpec((2, _BM_E, HALF), lambda i: (0, i, 0)),
        out_shape=jax.ShapeDtypeStruct((2, E, HALF), _F32),
    )(e2, w, b.reshape(1, H))


def _emb_e(ex, w, b):
    K = ex.shape[1]

    def body(x_ref, w_ref, b_ref, o_ref):
        acc = jnp.dot(x_ref[...], w_ref[...], preferred_element_type=_F32)
        acc = acc + b_ref[...]
        o_ref[0] = acc[:, 0:64]
        o_ref[1] = acc[:, 64:128]

    return pl.pallas_call(
        body,
        grid=(E // _BM_E,),
        in_specs=[
            pl.BlockSpec((_BM_E, K), lambda i: (i, 0)),
            pl.BlockSpec((K, H), lambda i: (0, 0)),
            pl.BlockSpec((1, H), lambda i: (0, 0)),
        ],
        out_specs=pl.BlockSpec((2, _BM_E, HALF), lambda i: (0, i, 0)),
        out_shape=jax.ShapeDtypeStruct((2, E, HALF), _F32),
    )(ex, w, b.reshape(1, H))


def _node_update(h_in, Ah, num2, den2, g, b):
    def body(h_ref, a_ref, n_ref, d_ref, g_ref, b_ref, o_ref):
        r0 = n_ref[0] / (d_ref[0] + 1e-6)
        r1 = n_ref[1] / (d_ref[1] + 1e-6)
        hn = a_ref[...] + jnp.concatenate([r0, r1], axis=1)
        hn = jnp.maximum(hn * g_ref[...] + b_ref[...], 0.0)
        o_ref[...] = h_ref[...] + hn

    return pl.pallas_call(
        body,
        grid=(N // _BM_N,),
        in_specs=[
            pl.BlockSpec((_BM_N, H), lambda i: (i, 0)),
            pl.BlockSpec((_BM_N, H), lambda i: (i, 0)),
            pl.BlockSpec((2, _BM_N, HALF), lambda i: (0, i, 0)),
            pl.BlockSpec((2, _BM_N, HALF), lambda i: (0, i, 0)),
            pl.BlockSpec((1, H), lambda i: (0, 0)),
            pl.BlockSpec((1, H), lambda i: (0, 0)),
        ],
        out_specs=pl.BlockSpec((_BM_N, H), lambda i: (i, 0)),
        out_shape=jax.ShapeDtypeStruct((N, H), _F32),
    )(h_in, Ah, num2, den2, g, b)


def _dec_emb(h1, lat, eps, wh, wz, b):
    def body(h_ref, l_ref, e_ref, wh_ref, wz_ref, b_ref, o_ref):
        mean = l_ref[:, 0:Z]
        logvar = l_ref[:, Z:2 * Z]
        z = e_ref[...] * jnp.exp(0.5 * logvar) + mean
        acc = jnp.dot(h_ref[...], wh_ref[...], preferred_element_type=_F32)
        acc = acc + jnp.dot(z, wz_ref[...], preferred_element_type=_F32)
        o_ref[...] = acc + b_ref[...]

    return pl.pallas_call(
        body,
        grid=(N // _BM_N,),
        in_specs=[
            pl.BlockSpec((_BM_N, H), lambda i: (i, 0)),
            pl.BlockSpec((_BM_N, 2 * Z), lambda i: (i, 0)),
            pl.BlockSpec((_BM_N, Z), lambda i: (i, 0)),
            pl.BlockSpec((H, H), lambda i: (0, 0)),
            pl.BlockSpec((Z, H), lambda i: (0, 0)),
            pl.BlockSpec((1, H), lambda i: (0, 0)),
        ],
        out_specs=pl.BlockSpec((_BM_N, H), lambda i: (i, 0)),
        out_shape=jax.ShapeDtypeStruct((N, H), _F32),
    )(h1, lat, eps, wh, wz, b.reshape(1, H))


def _readout(h, ro):
    w1, b1 = ro[0]["w"], ro[0]["b"]
    w2, b2 = ro[1]["w"], ro[1]["b"]
    w3, b3 = ro[2]["w"], ro[2]["b"]

    def body(x_ref, w1r, b1r, w2r, b2r, w3r, b3r, o_ref):
        y = jnp.dot(x_ref[...], w1r[...], preferred_element_type=_F32) + b1r[...]
        y = jnp.maximum(y, 0.0)
        y = jnp.dot(y, w2r[...], preferred_element_type=_F32) + b2r[...]
        y = jnp.maximum(y, 0.0)
        o_ref[...] = jnp.dot(y, w3r[...], preferred_element_type=_F32) + b3r[...]

    return pl.pallas_call(
        body,
        grid=(N // _BM_N,),
        in_specs=[
            pl.BlockSpec((_BM_N, H), lambda i: (i, 0)),
            pl.BlockSpec((H, 64), lambda i: (0, 0)),
            pl.BlockSpec((1, 64), lambda i: (0, 0)),
            pl.BlockSpec((64, 32), lambda i: (0, 0)),
            pl.BlockSpec((1, 32), lambda i: (0, 0)),
            pl.BlockSpec((32, 2), lambda i: (0, 0)),
            pl.BlockSpec((1, 2), lambda i: (0, 0)),
        ],
        out_specs=pl.BlockSpec((_BM_N, 2), lambda i: (i, 0)),
        out_shape=jax.ShapeDtypeStruct((N, 2), _F32),
    )(h, w1, b1.reshape(1, 64), w2, b2.reshape(1, 32), w3, b3.reshape(1, 2))


def _concat_e(e2):
    def body(x_ref, o_ref):
        o_ref[...] = jnp.concatenate([x_ref[0], x_ref[1]], axis=1)

    return pl.pallas_call(
        body,
        grid=(E // _BM_E,),
        in_specs=[pl.BlockSpec((2, _BM_E, HALF), lambda i: (0, i, 0))],
        out_specs=pl.BlockSpec((_BM_E, H), lambda i: (i, 0)),
        out_shape=jax.ShapeDtypeStruct((E, H), _F32),
    )(e2)


def _make_sc_edge():
    mesh = plsc.VectorSubcoreMesh(core_axis_name="c", subcore_axis_name="s",
                                  num_cores=_NSC, num_subcores=_NTEC)
    edge_buf = pltpu.VMEM((_C, HALF), _F32)

    @functools.partial(
        pl.kernel,
        out_type=[
            jax.ShapeDtypeStruct((2 * E, HALF), _F32),
            jax.ShapeDtypeStruct((2 * N, HALF), _F32),
            jax.ShapeDtypeStruct((2 * N, HALF), _F32),
        ],
        mesh=mesh,
        compiler_params=pltpu.CompilerParams(use_tc_tiling_on_sc=False),
        scratch_types=[
            pltpu.VMEM((2, _C), jnp.int32),
            pltpu.VMEM((2, _C), jnp.int32),
            pltpu.VMEM((2, _C), jnp.int32),
            edge_buf,
            edge_buf,
            edge_buf,
            edge_buf,
            edge_buf,
            pltpu.VMEM((HALF,), _F32),
            pltpu.VMEM((HALF,), _F32),
            pltpu.VMEM((_ZROWS, HALF), _F32),
            pltpu.VMEM_SHARED((N, HALF), _F32),
            pltpu.VMEM_SHARED((N, HALF), _F32),
            pltpu.SemaphoreType.DMA,
            pltpu.SemaphoreType.DMA,
            pltpu.SemaphoreType.DMA,
        ],
    )
    def sck(dtab_h, etab_h, btab_h, ce_h, ein_h, src_h, dst_h, g_h, b_h,
            eout_h, num_h, den_h,
            sidx, didx, dadj, dh, eh, bh, ce, ein,
            gv, bv, zbuf, num_sh, den_sh, sem, isem0, isem1):
        isem = (isem0, isem1)
        c = lax.axis_index("c")
        s = lax.axis_index("s")
        cn = c * N

        pltpu.sync_copy(g_h.at[pl.ds(c * HALF, HALF)], gv)
        pltpu.sync_copy(b_h.at[pl.ds(c * HALF, HALF)], bv)

        zv = jnp.zeros((16,), _F32)

        def zrow(r, carry):
            for j in range(HALF // 16):
                zbuf[r, pl.ds(j * 16, 16)] = zv
            return carry

        lax.fori_loop(0, _ZROWS, zrow, 0)

        @pl.when(s < _NCOPY)
        def _zero_stripes():
            for q in range(_NSTRIPE // _ZROWS):
                off = s * _NSTRIPE + q * _ZROWS
                pltpu.sync_copy(zbuf, num_sh.at[pl.ds(off, _ZROWS)])
                pltpu.sync_copy(zbuf, den_sh.at[pl.ds(off, _ZROWS)])

        plsc.subcore_barrier()

        ebase = s * _EPT

        def issue_idx(k, r):
            base = ebase + k * _C
            pltpu.async_copy(src_h.at[pl.ds(base, _C)], sidx.at[r], isem[r])
            pltpu.async_copy(dst_h.at[pl.ds(base, _C)], didx.at[r], isem[r])

        def wait_idx_adjust(r):
            pltpu.make_async_copy(
                src_h.at[pl.ds(0, _C)], sidx.at[r], isem[r]).wait()
            pltpu.make_async_copy(
                dst_h.at[pl.ds(0, _C)], didx.at[r], isem[r]).wait()
            for j in range(_C // 16):
                sl = pl.ds(j * 16, 16)
                sidx[r, sl] = sidx[r, sl] + cn
                dadj[r, sl] = didx[r, sl] + cn

        def do_chunk(k, r):
            base = ebase + k * _C
            d1 = pltpu.async_copy(dtab_h.at[sidx.at[r]], dh, sem)
            d2 = pltpu.async_copy(etab_h.at[dadj.at[r]], eh, sem)
            d3 = pltpu.async_copy(btab_h.at[sidx.at[r]], bh, sem)
            fb = c * E + base
            d4 = pltpu.async_copy(ce_h.at[pl.ds(fb, _C)], ce, sem)
            d5 = pltpu.async_copy(ein_h.at[pl.ds(fb, _C)], ein, sem)
            d1.wait()
            d2.wait()
            d3.wait()
            d4.wait()
            d5.wait()

            def row(r_, rc):
                for j in range(HALF // 16):
                    sl = pl.ds(j * 16, 16)
                    en = dh[r_, sl] + eh[r_, sl] + ce[r_, sl]
                    sgv = 1.0 / (1.0 + jnp.exp(-en))
                    eov = jnp.maximum(en * gv[sl] + bv[sl], 0.0) + ein[r_, sl]
                    dh[r_, sl] = sgv
                    bh[r_, sl] = sgv * bh[r_, sl]
                    eh[r_, sl] = eov
                return rc

            lax.fori_loop(0, _C, row, 0)
            pltpu.sync_copy(eh, eout_h.at[pl.ds(fb, _C)])
            pltpu.sync_copy(bh, num_sh.at[didx.at[r]], add=True)
            pltpu.sync_copy(dh, den_sh.at[didx.at[r]], add=True)

        issue_idx(0, 0)

        def dbl(i, carry):
            k0 = 2 * i
            wait_idx_adjust(0)
            issue_idx(k0 + 1, 1)
            do_chunk(k0, 0)
            wait_idx_adjust(1)
            issue_idx(k0 + 2, 0)
            do_chunk(k0 + 1, 1)
            return carry

        lax.fori_loop(0, (_NCH - 1) // 2, dbl, 0)

        wait_idx_adjust(0)
        do_chunk(_NCH - 1, 0)

        plsc.subcore_barrier()

        @pl.when(s < _NCOPY)
        def _copy_out():
            nb = cn + s * _NSTRIPE
            pltpu.sync_copy(num_sh.at[pl.ds(s * _NSTRIPE, _NSTRIPE)],
                            num_h.at[pl.ds(nb, _NSTRIPE)])
            pltpu.sync_copy(den_sh.at[pl.ds(s * _NSTRIPE, _NSTRIPE)],
                            den_h.at[pl.ds(nb, _NSTRIPE)])

    return sck


_SC_EDGE = _make_sc_edge()


def _sc_edge(D2, E2t, B2, Ce2, ein2, src, dst, g, b):
    eo, nu, de = _SC_EDGE(
        D2.reshape(2 * N, HALF), E2t.reshape(2 * N, HALF),
        B2.reshape(2 * N, HALF), Ce2.reshape(2 * E, HALF),
        ein2.reshape(2 * E, HALF), src, dst, g, b)
    return (eo.reshape(2, E, HALF), nu.reshape(2, N, HALF),
            de.reshape(2, N, HALF))


_BN_SCALE = 1.0 / np.sqrt(1.0 + 1e-5)


def _layer(h, e2, src, dst, lp):
    w_all = jnp.concatenate(
        [lp["A"]["w"], lp["B"]["w"], lp["D"]["w"], lp["E"]["w"]], axis=1)
    b_all = jnp.concatenate(
        [lp["A"]["b"], lp["B"]["b"], lp["D"]["b"], lp["E"]["b"]]).reshape(1, 512)
    Ah, B2, D2, E2t = _abde(h, w_all, b_all)
    Ce2 = _ce(e2, lp["C"]["w"], lp["C"]["b"])
    ge = lp["bn_e_g"] * _BN_SCALE
    eout2, num2, den2 = _sc_edge(D2, E2t, B2, Ce2, e2, src, dst,
                                 ge, lp["bn_e_b"])
    gh = (lp["bn_h_g"] * _BN_SCALE).reshape(1, H)
    h = _node_update(h, Ah, num2, den2, gh, lp["bn_h_b"].reshape(1, H))
    return h, eout2


def kernel(xx, ex, edge_index, params):
    src = edge_index[0]
    dst = edge_index[1]
    eps = jax.random.normal(jax.random.key(1), (N, Z), dtype=_F32)

    penc = params["past_enc"]
    pdec = params["past_dec"]

    h = _linear_dense(xx, penc["emb_h"]["w"], penc["emb_h"]["b"],
                      block_m=_BM_N)
    e2 = _emb_e(ex, penc["emb_e"]["w"], penc["emb_e"]["b"])
    for lp in penc["layers"]:
        h, e2 = _layer(h, e2, src, dst, lp)

    lat = _linear_dense(h, params["prior"]["w"], params["prior"]["b"],
                        block_m=_BM_N)
    h = _dec_emb(h, lat, eps, pdec["emb_h"]["w"][:H], pdec["emb_h"]["w"][H:],
                 pdec["emb_h"]["b"])
    e2 = _ce(e2, pdec["emb_e"]["w"], pdec["emb_e"]["b"])
    for lp in pdec["layers"]:
        h, e2 = _layer(h, e2, src, dst, lp)

    h_out = _readout(h, pdec["readout"])
    e_out = _concat_e(e2)
    return h_out, e_out


# speedup vs baseline: 1.5438x; 1.1633x over previous
import functools

import jax
import jax.numpy as jnp
import numpy as np
from jax import lax
from jax.experimental import pallas as pl
from jax.experimental.pallas import tpu as pltpu
from jax.experimental.pallas import tpu_sc as plsc

N = 10000
E = 160000
H = 128
HALF = 64
Z = 32

_BM_N = 2000
_BM_E = 2000

_NSC = 2
_NTEC = 16
_EPT = E // _NTEC
_C = 80
_NCH = _EPT // _C
_NCOPY = 10
_NSTRIPE = N // _NCOPY
_ZROWS = 200

_F32 = jnp.float32


def _linear_dense(x, w, b, *, relu=False, block_m):
    M, K = x.shape
    F = w.shape[1]

    def body(x_ref, w_ref, b_ref, o_ref):
        acc = jnp.dot(x_ref[...], w_ref[...], preferred_element_type=_F32)
        acc = acc + b_ref[...]
        if relu:
            acc = jnp.maximum(acc, 0.0)
        o_ref[...] = acc

    return pl.pallas_call(
        body,
        grid=(M // block_m,),
        in_specs=[
            pl.BlockSpec((block_m, K), lambda i: (i, 0)),
            pl.BlockSpec((K, F), lambda i: (0, 0)),
            pl.BlockSpec((1, F), lambda i: (0, 0)),
        ],
        out_specs=pl.BlockSpec((block_m, F), lambda i: (i, 0)),
        out_shape=jax.ShapeDtypeStruct((M, F), _F32),
    )(x, w, b.reshape(1, F))


def _abde(h, w_all, b_all):
    def body(x_ref, w_ref, b_ref, a_ref, b2_ref, d2_ref, e2_ref):
        acc = jnp.dot(x_ref[...], w_ref[...], preferred_element_type=_F32)
        acc = acc + b_ref[...]
        a_ref[...] = acc[:, 0:128]
        for t, ref in enumerate((b2_ref, d2_ref, e2_ref)):
            off = 128 * (t + 1)
            ref[0] = acc[:, off:off + 64]
            ref[1] = acc[:, off + 64:off + 128]

    return pl.pallas_call(
        body,
        grid=(N // _BM_N,),
        in_specs=[
            pl.BlockSpec((_BM_N, H), lambda i: (i, 0)),
            pl.BlockSpec((H, 512), lambda i: (0, 0)),
            pl.BlockSpec((1, 512), lambda i: (0, 0)),
        ],
        out_specs=[
            pl.BlockSpec((_BM_N, H), lambda i: (i, 0)),
            pl.BlockSpec((2, _BM_N, HALF), lambda i: (0, i, 0)),
            pl.BlockSpec((2, _BM_N, HALF), lambda i: (0, i, 0)),
            pl.BlockSpec((2, _BM_N, HALF), lambda i: (0, i, 0)),
        ],
        out_shape=[
            jax.ShapeDtypeStruct((N, H), _F32),
            jax.ShapeDtypeStruct((2, N, HALF), _F32),
            jax.ShapeDtypeStruct((2, N, HALF), _F32),
            jax.ShapeDtypeStruct((2, N, HALF), _F32),
        ],
    )(h, w_all, b_all)


def _ce(e2, w, b):
    def body(x_ref, w_ref, b_ref, o_ref):
        x = jnp.concatenate([x_ref[0], x_ref[1]], axis=1)
        acc = jnp.dot(x, w_ref[...], preferred_element_type=_F32)
        acc = acc + b_ref[...]
        o_ref[0] = acc[:, 0:64]
        o_ref[1] = acc[:, 64:128]

    return pl.pallas_call(
        body,
        grid=(E // _BM_E,),
        in_specs=[
            pl.BlockSpec((2, _BM_E, HALF), lambda i: (0, i, 0)),
            pl.BlockSpec((H, H), lambda i: (0, 0)),
            pl.BlockSpec((1, H), lambda i: (0, 0)),
        ],
        out_specs=pl.BlockSpec((2, _BM_E, HALF), lambda i: (0, i, 0)),
        out_shape=jax.ShapeDtypeStruct((2, E, HALF), _F32),
    )(e2, w, b.reshape(1, H))


def _emb_e(ex, w, b):
    K = ex.shape[1]

    def body(x_ref, w_ref, b_ref, o_ref):
        acc = jnp.dot(x_ref[...], w_ref[...], preferred_element_type=_F32)
        acc = acc + b_ref[...]
        o_ref[0] = acc[:, 0:64]
        o_ref[1] = acc[:, 64:128]

    return pl.pallas_call(
        body,
        grid=(E // _BM_E,),
        in_specs=[
            pl.BlockSpec((_BM_E, K), lambda i: (i, 0)),
            pl.BlockSpec((K, H), lambda i: (0, 0)),
            pl.BlockSpec((1, H), lambda i: (0, 0)),
        ],
        out_specs=pl.BlockSpec((2, _BM_E, HALF), lambda i: (0, i, 0)),
        out_shape=jax.ShapeDtypeStruct((2, E, HALF), _F32),
    )(ex, w, b.reshape(1, H))


def _node_update(h_in, Ah, num2, den2, g, b):
    def body(h_ref, a_ref, n_ref, d_ref, g_ref, b_ref, o_ref):
        r0 = n_ref[0] / (d_ref[0] + 1e-6)
        r1 = n_ref[1] / (d_ref[1] + 1e-6)
        hn = a_ref[...] + jnp.concatenate([r0, r1], axis=1)
        hn = jnp.maximum(hn * g_ref[...] + b_ref[...], 0.0)
        o_ref[...] = h_ref[...] + hn

    return pl.pallas_call(
        body,
        grid=(N // _BM_N,),
        in_specs=[
            pl.BlockSpec((_BM_N, H), lambda i: (i, 0)),
            pl.BlockSpec((_BM_N, H), lambda i: (i, 0)),
            pl.BlockSpec((2, _BM_N, HALF), lambda i: (0, i, 0)),
            pl.BlockSpec((2, _BM_N, HALF), lambda i: (0, i, 0)),
            pl.BlockSpec((1, H), lambda i: (0, 0)),
            pl.BlockSpec((1, H), lambda i: (0, 0)),
        ],
        out_specs=pl.BlockSpec((_BM_N, H), lambda i: (i, 0)),
        out_shape=jax.ShapeDtypeStruct((N, H), _F32),
    )(h_in, Ah, num2, den2, g, b)


def _dec_emb(h1, lat, eps, wh, wz, b):
    def body(h_ref, l_ref, e_ref, wh_ref, wz_ref, b_ref, o_ref):
        mean = l_ref[:, 0:Z]
        logvar = l_ref[:, Z:2 * Z]
        z = e_ref[...] * jnp.exp(0.5 * logvar) + mean
        acc = jnp.dot(h_ref[...], wh_ref[...], preferred_element_type=_F32)
        acc = acc + jnp.dot(z, wz_ref[...], preferred_element_type=_F32)
        o_ref[...] = acc + b_ref[...]

    return pl.pallas_call(
        body,
        grid=(N // _BM_N,),
        in_specs=[
            pl.BlockSpec((_BM_N, H), lambda i: (i, 0)),
            pl.BlockSpec((_BM_N, 2 * Z), lambda i: (i, 0)),
            pl.BlockSpec((_BM_N, Z), lambda i: (i, 0)),
            pl.BlockSpec((H, H), lambda i: (0, 0)),
            pl.BlockSpec((Z, H), lambda i: (0, 0)),
            pl.BlockSpec((1, H), lambda i: (0, 0)),
        ],
        out_specs=pl.BlockSpec((_BM_N, H), lambda i: (i, 0)),
        out_shape=jax.ShapeDtypeStruct((N, H), _F32),
    )(h1, lat, eps, wh, wz, b.reshape(1, H))


def _readout(h, ro):
    w1, b1 = ro[0]["w"], ro[0]["b"]
    w2, b2 = ro[1]["w"], ro[1]["b"]
    w3, b3 = ro[2]["w"], ro[2]["b"]

    def body(x_ref, w1r, b1r, w2r, b2r, w3r, b3r, o_ref):
        y = jnp.dot(x_ref[...], w1r[...], preferred_element_type=_F32) + b1r[...]
        y = jnp.maximum(y, 0.0)
        y = jnp.dot(y, w2r[...], preferred_element_type=_F32) + b2r[...]
        y = jnp.maximum(y, 0.0)
        o_ref[...] = jnp.dot(y, w3r[...], preferred_element_type=_F32) + b3r[...]

    return pl.pallas_call(
        body,
        grid=(N // _BM_N,),
        in_specs=[
            pl.BlockSpec((_BM_N, H), lambda i: (i, 0)),
            pl.BlockSpec((H, 64), lambda i: (0, 0)),
            pl.BlockSpec((1, 64), lambda i: (0, 0)),
            pl.BlockSpec((64, 32), lambda i: (0, 0)),
            pl.BlockSpec((1, 32), lambda i: (0, 0)),
            pl.BlockSpec((32, 2), lambda i: (0, 0)),
            pl.BlockSpec((1, 2), lambda i: (0, 0)),
        ],
        out_specs=pl.BlockSpec((_BM_N, 2), lambda i: (i, 0)),
        out_shape=jax.ShapeDtypeStruct((N, 2), _F32),
    )(h, w1, b1.reshape(1, 64), w2, b2.reshape(1, 32), w3, b3.reshape(1, 2))


def _concat_e(e2):
    def body(x_ref, o_ref):
        o_ref[...] = jnp.concatenate([x_ref[0], x_ref[1]], axis=1)

    return pl.pallas_call(
        body,
        grid=(E // _BM_E,),
        in_specs=[pl.BlockSpec((2, _BM_E, HALF), lambda i: (0, i, 0))],
        out_specs=pl.BlockSpec((_BM_E, H), lambda i: (i, 0)),
        out_shape=jax.ShapeDtypeStruct((E, H), _F32),
    )(e2)


def _make_sc_edge():
    mesh = plsc.VectorSubcoreMesh(core_axis_name="c", subcore_axis_name="s",
                                  num_cores=_NSC, num_subcores=_NTEC)
    edge_buf = pltpu.VMEM((_C, HALF), _F32)

    @functools.partial(
        pl.kernel,
        out_type=[
            jax.ShapeDtypeStruct((2 * E, HALF), _F32),
            jax.ShapeDtypeStruct((2 * N, HALF), _F32),
            jax.ShapeDtypeStruct((2 * N, HALF), _F32),
        ],
        mesh=mesh,
        compiler_params=pltpu.CompilerParams(use_tc_tiling_on_sc=False),
        scratch_types=[
            pltpu.VMEM((2, _C), jnp.int32),
            pltpu.VMEM((2, _C), jnp.int32),
            pltpu.VMEM((2, _C), jnp.int32),
            edge_buf,
            edge_buf,
            edge_buf,
            edge_buf,
            edge_buf,
            pltpu.VMEM((HALF,), _F32),
            pltpu.VMEM((HALF,), _F32),
            pltpu.VMEM((_ZROWS, HALF), _F32),
            pltpu.VMEM_SHARED((N, HALF), _F32),
            pltpu.VMEM_SHARED((N, HALF), _F32),
            pltpu.SemaphoreType.DMA,
            pltpu.SemaphoreType.DMA,
            pltpu.SemaphoreType.DMA,
        ],
    )
    def sck(dtab_h, etab_h, btab_h, ce_h, ein_h, src_h, dst_h, g_h, b_h,
            eout_h, num_h, den_h,
            sidx, didx, dadj, dh, eh, bh, ce, ein,
            gv, bv, zbuf, num_sh, den_sh, sem, isem0, isem1):
        isem = (isem0, isem1)
        c = lax.axis_index("c")
        s = lax.axis_index("s")
        cn = c * N

        pltpu.sync_copy(g_h.at[pl.ds(c * HALF, HALF)], gv)
        pltpu.sync_copy(b_h.at[pl.ds(c * HALF, HALF)], bv)

        zv = jnp.zeros((16,), _F32)

        def zrow(r, carry):
            for j in range(HALF // 16):
                zbuf[r, pl.ds(j * 16, 16)] = zv
            return carry

        lax.fori_loop(0, _ZROWS, zrow, 0)

        @pl.when(s < _NCOPY)
        def _zero_stripes():
            for q in range(_NSTRIPE // _ZROWS):
                off = s * _NSTRIPE + q * _ZROWS
                pltpu.sync_copy(zbuf, num_sh.at[pl.ds(off, _ZROWS)])
                pltpu.sync_copy(zbuf, den_sh.at[pl.ds(off, _ZROWS)])

        plsc.subcore_barrier()

        ebase = s * _EPT

        def issue_idx(k, r):
            base = ebase + k * _C
            pltpu.async_copy(src_h.at[pl.ds(base, _C)], sidx.at[r], isem[r])
            pltpu.async_copy(dst_h.at[pl.ds(base, _C)], didx.at[r], isem[r])

        def wait_idx_adjust(r):
            pltpu.make_async_copy(
                src_h.at[pl.ds(0, _C)], sidx.at[r], isem[r]).wait()
            pltpu.make_async_copy(
                dst_h.at[pl.ds(0, _C)], didx.at[r], isem[r]).wait()
            for j in range(_C // 16):
                sl = pl.ds(j * 16, 16)
                sidx[r, sl] = sidx[r, sl] + cn
                dadj[r, sl] = didx[r, sl] + cn

        def do_chunk(k, r):
            base = ebase + k * _C
            d1 = pltpu.async_copy(dtab_h.at[sidx.at[r]], dh, sem)
            d2 = pltpu.async_copy(etab_h.at[dadj.at[r]], eh, sem)
            d3 = pltpu.async_copy(btab_h.at[sidx.at[r]], bh, sem)
            fb = c * E + base
            d4 = pltpu.async_copy(ce_h.at[pl.ds(fb, _C)], ce, sem)
            d5 = pltpu.async_copy(ein_h.at[pl.ds(fb, _C)], ein, sem)
            d1.wait()
            d2.wait()
            d3.wait()
            d4.wait()
            d5.wait()

            gs = tuple(gv[pl.ds(j * 16, 16)] for j in range(HALF // 16))
            bs = tuple(bv[pl.ds(j * 16, 16)] for j in range(HALF // 16))

            @plsc.parallel_loop(0, _C, 1, unroll=4)
            def row(r_):
                for j in range(HALF // 16):
                    sl = pl.ds(j * 16, 16)
                    en = dh[r_, sl] + eh[r_, sl] + ce[r_, sl]
                    sgv = 1.0 / (1.0 + jnp.exp(-en))
                    eov = jnp.maximum(en * gs[j] + bs[j], 0.0) + ein[r_, sl]
                    dh[r_, sl] = sgv
                    bh[r_, sl] = sgv * bh[r_, sl]
                    eh[r_, sl] = eov
            pltpu.sync_copy(eh, eout_h.at[pl.ds(fb, _C)])
            pltpu.sync_copy(bh, num_sh.at[didx.at[r]], add=True)
            pltpu.sync_copy(dh, den_sh.at[didx.at[r]], add=True)

        issue_idx(0, 0)

        def dbl(i, carry):
            k0 = 2 * i
            wait_idx_adjust(0)
            issue_idx(k0 + 1, 1)
            do_chunk(k0, 0)
            wait_idx_adjust(1)
            issue_idx(k0 + 2, 0)
            do_chunk(k0 + 1, 1)
            return carry

        lax.fori_loop(0, (_NCH - 1) // 2, dbl, 0)

        wait_idx_adjust(0)
        do_chunk(_NCH - 1, 0)

        plsc.subcore_barrier()

        @pl.when(s < _NCOPY)
        def _copy_out():
            nb = cn + s * _NSTRIPE
            pltpu.sync_copy(num_sh.at[pl.ds(s * _NSTRIPE, _NSTRIPE)],
                            num_h.at[pl.ds(nb, _NSTRIPE)])
            pltpu.sync_copy(den_sh.at[pl.ds(s * _NSTRIPE, _NSTRIPE)],
                            den_h.at[pl.ds(nb, _NSTRIPE)])

    return sck


_SC_EDGE = _make_sc_edge()


def _sc_edge(D2, E2t, B2, Ce2, ein2, src, dst, g, b):
    eo, nu, de = _SC_EDGE(
        D2.reshape(2 * N, HALF), E2t.reshape(2 * N, HALF),
        B2.reshape(2 * N, HALF), Ce2.reshape(2 * E, HALF),
        ein2.reshape(2 * E, HALF), src, dst, g, b)
    return (eo.reshape(2, E, HALF), nu.reshape(2, N, HALF),
            de.reshape(2, N, HALF))


_BN_SCALE = 1.0 / np.sqrt(1.0 + 1e-5)


def _layer(h, e2, src, dst, lp):
    w_all = jnp.concatenate(
        [lp["A"]["w"], lp["B"]["w"], lp["D"]["w"], lp["E"]["w"]], axis=1)
    b_all = jnp.concatenate(
        [lp["A"]["b"], lp["B"]["b"], lp["D"]["b"], lp["E"]["b"]]).reshape(1, 512)
    Ah, B2, D2, E2t = _abde(h, w_all, b_all)
    Ce2 = _ce(e2, lp["C"]["w"], lp["C"]["b"])
    ge = lp["bn_e_g"] * _BN_SCALE
    eout2, num2, den2 = _sc_edge(D2, E2t, B2, Ce2, e2, src, dst,
                                 ge, lp["bn_e_b"])
    gh = (lp["bn_h_g"] * _BN_SCALE).reshape(1, H)
    h = _node_update(h, Ah, num2, den2, gh, lp["bn_h_b"].reshape(1, H))
    return h, eout2


def kernel(xx, ex, edge_index, params):
    src = edge_index[0]
    dst = edge_index[1]
    eps = jax.random.normal(jax.random.key(1), (N, Z), dtype=_F32)

    penc = params["past_enc"]
    pdec = params["past_dec"]

    h = _linear_dense(xx, penc["emb_h"]["w"], penc["emb_h"]["b"],
                      block_m=_BM_N)
    e2 = _emb_e(ex, penc["emb_e"]["w"], penc["emb_e"]["b"])
    for lp in penc["layers"]:
        h, e2 = _layer(h, e2, src, dst, lp)

    lat = _linear_dense(h, params["prior"]["w"], params["prior"]["b"],
                        block_m=_BM_N)
    h = _dec_emb(h, lat, eps, pdec["emb_h"]["w"][:H], pdec["emb_h"]["w"][H:],
                 pdec["emb_h"]["b"])
    e2 = _ce(e2, pdec["emb_e"]["w"], pdec["emb_e"]["b"])
    for lp in pdec["layers"]:
        h, e2 = _layer(h, e2, src, dst, lp)

    h_out = _readout(h, pdec["readout"])
    e_out = _concat_e(e2)
    return h_out, e_out
